# baseline (speedup 1.0000x reference)
import functools

import jax
import jax.numpy as jnp
from jax import lax
from jax.experimental import pallas as pl
from jax.experimental.pallas import tpu as pltpu
from jax.experimental.pallas import tpu_sc as plsc

N = 10000
E = 320000
D = 128
NC, NS, L = 2, 16, 16
NW = NC * NS
EPT = E // NW
CH = 80
NCHK = EPT // CH
VPC = CH // L
NP = 10240
RPT = NP // NS
NPR = 10112
RPTR = NPR // NS

_MESH = plsc.VectorSubcoreMesh(
    core_axis_name="c", subcore_axis_name="s", num_cores=NC, num_subcores=NS)
_SC_PARAMS = pltpu.CompilerParams(needs_layout_passes=False)


def _lrelu(v):
    return jnp.where(v >= 0, v, 0.2 * v)


def _bcast_lane(vec, u):
    idx = jnp.full((L,), u, jnp.int32)
    return lax.gather(vec, idx[:, None],
                      dimension_numbers=lax.GatherDimensionNumbers(
                          offset_dims=(), collapsed_slice_dims=(0,),
                          start_index_map=(0,)),
                      slice_sizes=(1,),
                      mode=lax.GatherScatterMode.PROMISE_IN_BOUNDS)


def _tc_pre_body(x_ref, w_ref, asr_ref, adr_ref, h_ref, as_ref, ad_ref, m_ref):
    h = jnp.dot(x_ref[...], w_ref[...], preferred_element_type=jnp.float32)
    h_ref[...] = h
    a_s = jnp.sum(h * asr_ref[...][None, :], axis=1)
    a_d = jnp.sum(h * adr_ref[...][None, :], axis=1)
    pad = jnp.full((NP - N,), -1e30, jnp.float32)
    as_ref[...] = jnp.concatenate([a_s, pad])
    ad_ref[...] = jnp.concatenate([a_d, pad])
    m_ref[...] = jnp.full((128,), _lrelu(jnp.max(a_s) + jnp.max(a_d)),
                          jnp.float32)


_tc_pre = pl.pallas_call(
    _tc_pre_body,
    out_shape=[
        jax.ShapeDtypeStruct((N, D), jnp.float32),
        jax.ShapeDtypeStruct((NP,), jnp.float32),
        jax.ShapeDtypeStruct((NP,), jnp.float32),
        jax.ShapeDtypeStruct((128,), jnp.float32),
    ],
)


CH = 80
GG = 25
NG = NCHK // GG
NPAIR = (GG - 1) // 2
NFULL = RPTR // CH


def _sc_main_body(h_hbm, as_hbm, ad_hbm, m_hbm, sidx_hbm, didx_hbm,
                  dcat_hbm, pcat_hbm,
                  asv, adv, sxg, dxg, exc, mv, g0, g1, sem0, sem1, dsh, osh):
    cid = lax.axis_index("c")
    sid = lax.axis_index("s")
    wid = cid * NS + sid

    pltpu.sync_copy(as_hbm.at[pl.ds(0, N)], asv)
    pltpu.sync_copy(ad_hbm.at[pl.ds(0, N)], adv)
    pltpu.sync_copy(m_hbm.at[pl.ds(0, 16)], mv)

    def _zg(r, _):
        for q in range(D // L):
            g0[r, pl.ds(q * L, L)] = jnp.zeros((L,), jnp.float32)
        return 0

    lax.fori_loop(0, CH, _zg, 0)
    base = sid * RPTR
    for j in range(NFULL):
        pltpu.sync_copy(g0, osh.at[pl.ds(base + j * CH, CH)])
    pltpu.sync_copy(g0.at[pl.ds(0, RPTR - NFULL * CH)],
                    osh.at[pl.ds(base + NFULL * CH, RPTR - NFULL * CH)])
    for j in range(RPT // 128):
        pltpu.sync_copy(g0.at[0, pl.ds(0, D)],
                        dsh.at[pl.ds(sid * RPT + j * 128, 128)])
    plsc.subcore_barrier()

    mval = mv[...]

    def _do_chunk(k, gp):
        @plsc.parallel_loop(0, VPC, unroll=VPC)
        def _vv(v):
            s_ids = sxg[k, pl.ds(v * L, L)]
            d_ids = dxg[k, pl.ds(v * L, L)]
            a = plsc.load_gather(asv, [s_ids]) + plsc.load_gather(adv, [d_ids])
            ex = jnp.exp(_lrelu(a) - mval)
            exc[pl.ds(v * L, L)] = ex
            for u in range(L):
                r = v * L + u
                av = _bcast_lane(ex, u)
                for q in range(D // L):
                    gp[r, pl.ds(q * L, L)] = gp[r, pl.ds(q * L, L)] * av
        pltpu.sync_copy(exc, dsh.at[dxg.at[k]], add=True)
        pltpu.sync_copy(gp, osh.at[dxg.at[k]], add=True)

    def _wait(gp, sem):
        pltpu.make_async_copy(h_hbm.at[pl.ds(0, CH)], gp, sem).wait()

    def _group(g, _):
        pltpu.sync_copy(sidx_hbm.at[g, wid], sxg)
        pltpu.sync_copy(didx_hbm.at[g, wid], dxg)
        pltpu.async_copy(h_hbm.at[sxg.at[0]], g0, sem0)

        def _pair(j, _):
            c0 = 2 * j
            pltpu.async_copy(h_hbm.at[sxg.at[c0 + 1]], g1, sem1)
            _wait(g0, sem0)
            _do_chunk(c0, g0)
            pltpu.async_copy(h_hbm.at[sxg.at[c0 + 2]], g0, sem0)
            _wait(g1, sem1)
            _do_chunk(c0 + 1, g1)
            return 0

        lax.fori_loop(0, NPAIR, _pair, 0)
        _wait(g0, sem0)
        _do_chunk(GG - 1, g0)
        return 0

    lax.fori_loop(0, NG, _group, 0)
    plsc.subcore_barrier()

    pltpu.sync_copy(dsh.at[pl.ds(sid * RPT, RPT)],
                    dcat_hbm.at[pl.ds(cid * NP + sid * RPT, RPT)])
    for j in range(NFULL):
        pltpu.sync_copy(osh.at[pl.ds(base + j * CH, CH)],
                        pcat_hbm.at[pl.ds(cid * NPR + base + j * CH, CH)])
    pltpu.sync_copy(
        osh.at[pl.ds(base + NFULL * CH, RPTR - NFULL * CH)],
        pcat_hbm.at[pl.ds(cid * NPR + base + NFULL * CH, RPTR - NFULL * CH)])


_sc_main = pl.kernel(
    _sc_main_body,
    out_type=[
        jax.ShapeDtypeStruct((2 * NP,), jnp.float32),
        jax.ShapeDtypeStruct((2 * NPR, D), jnp.float32),
    ],
    mesh=_MESH,
    compiler_params=_SC_PARAMS,
    scratch_types=[
        pltpu.VMEM((N,), jnp.float32),
        pltpu.VMEM((N,), jnp.float32),
        pltpu.VMEM((GG, CH), jnp.int32),
        pltpu.VMEM((GG, CH), jnp.int32),
        pltpu.VMEM((CH,), jnp.float32),
        pltpu.VMEM((16,), jnp.float32),
        pltpu.VMEM((CH, D), jnp.float32),
        pltpu.VMEM((CH, D), jnp.float32),
        pltpu.SemaphoreType.DMA,
        pltpu.SemaphoreType.DMA,
        pltpu.VMEM_SHARED((NP,), jnp.float32),
        pltpu.VMEM_SHARED((NPR, D), jnp.float32),
    ],
)


def _tc_post_body(pcat_ref, h_ref, as_ref, ad_ref, m_ref, dcat_ref,
                  b_ref, out_ref):
    a = _lrelu(as_ref[0:N] + ad_ref[0:N])
    se = jnp.exp(a - m_ref[0])
    dn = dcat_ref[0:N] + dcat_ref[NP:NP + N] + se
    num = (pcat_ref[0:N, :] + pcat_ref[NPR:NPR + N, :]
           + se[:, None] * h_ref[...])
    out_ref[...] = num / (dn + 1e-16)[:, None] + b_ref[...][None, :]


_tc_post = pl.pallas_call(
    _tc_post_body,
    out_shape=jax.ShapeDtypeStruct((N, D), jnp.float32),
)


def kernel(x, edge_index, edge_attr, W, att_src, att_dst, bias,
           edge_emb_weight):
    srcf = edge_index[0].astype(jnp.int32)
    dstf = edge_index[1].astype(jnp.int32)
    h, a_s, a_d, m = _tc_pre(x, W, att_src, att_dst)
    dcat, pcat = _sc_main(h, a_s, a_d, m,
                          srcf.reshape(NG, NW, GG, CH),
                          dstf.reshape(NG, NW, GG, CH))
    return _tc_post(pcat, h, a_s, a_d, m, dcat, bias)


# speedup vs baseline: 1.0713x; 1.0713x over previous
import functools

import jax
import jax.numpy as jnp
from jax import lax
from jax.experimental import pallas as pl
from jax.experimental.pallas import tpu as pltpu
from jax.experimental.pallas import tpu_sc as plsc

N = 10000
E = 320000
D = 128
NC, NS, L = 2, 16, 16
NW = NC * NS
EPT = E // NW
CH = 80
NCHK = EPT // CH
VPC = CH // L
NP = 10240
RPT = NP // NS
NPR = 10112
RPTR = NPR // NS

_MESH = plsc.VectorSubcoreMesh(
    core_axis_name="c", subcore_axis_name="s", num_cores=NC, num_subcores=NS)
_SC_PARAMS = pltpu.CompilerParams(needs_layout_passes=False)


def _lrelu(v):
    return jnp.where(v >= 0, v, 0.2 * v)


def _bcast_lane(vec, u):
    idx = jnp.full((L,), u, jnp.int32)
    return lax.gather(vec, idx[:, None],
                      dimension_numbers=lax.GatherDimensionNumbers(
                          offset_dims=(), collapsed_slice_dims=(0,),
                          start_index_map=(0,)),
                      slice_sizes=(1,),
                      mode=lax.GatherScatterMode.PROMISE_IN_BOUNDS)


def _tc_pre_body(x_ref, w_ref, asr_ref, adr_ref, h_ref, as_ref, ad_ref, m_ref):
    h = jnp.dot(x_ref[...], w_ref[...], preferred_element_type=jnp.float32)
    h_ref[...] = h
    a_s = jnp.sum(h * asr_ref[...][None, :], axis=1)
    a_d = jnp.sum(h * adr_ref[...][None, :], axis=1)
    pad = jnp.full((NP - N,), -1e30, jnp.float32)
    as_ref[...] = jnp.concatenate([a_s, pad])
    ad_ref[...] = jnp.concatenate([a_d, pad])
    m_ref[...] = jnp.full((128,), _lrelu(jnp.max(a_s) + jnp.max(a_d)),
                          jnp.float32)


_tc_pre = pl.pallas_call(
    _tc_pre_body,
    out_shape=[
        jax.ShapeDtypeStruct((N, D), jnp.float32),
        jax.ShapeDtypeStruct((NP,), jnp.float32),
        jax.ShapeDtypeStruct((NP,), jnp.float32),
        jax.ShapeDtypeStruct((128,), jnp.float32),
    ],
)


CH = 80
GG = 25
NG = NCHK // GG
NPAIR = (GG - 1) // 2
NFULL = RPTR // CH


def _sc_main_body(h_hbm, as_hbm, ad_hbm, m_hbm, sidx_hbm, didx_hbm,
                  dcat_hbm, pcat_hbm,
                  asv, adv, sxg, dxg, exc, mv, g0, g1, sem0, sem1, dsh, osh):
    cid = lax.axis_index("c")
    sid = lax.axis_index("s")
    wid = cid * NS + sid

    pltpu.sync_copy(as_hbm.at[pl.ds(0, N)], asv)
    pltpu.sync_copy(ad_hbm.at[pl.ds(0, N)], adv)
    pltpu.sync_copy(m_hbm.at[pl.ds(0, 16)], mv)

    def _zg(r, _):
        for q in range(D // L):
            g0[r, pl.ds(q * L, L)] = jnp.zeros((L,), jnp.float32)
        return 0

    lax.fori_loop(0, CH, _zg, 0)
    base = sid * RPTR
    for j in range(NFULL):
        pltpu.sync_copy(g0, osh.at[pl.ds(base + j * CH, CH)])
    pltpu.sync_copy(g0.at[pl.ds(0, RPTR - NFULL * CH)],
                    osh.at[pl.ds(base + NFULL * CH, RPTR - NFULL * CH)])
    for j in range(RPT // 128):
        pltpu.sync_copy(g0.at[0, pl.ds(0, D)],
                        dsh.at[pl.ds(sid * RPT + j * 128, 128)])
    plsc.subcore_barrier()

    mval = mv[...]

    def _do_chunk(k, gp):
        @plsc.parallel_loop(0, VPC)
        def _vv(v):
            s_ids = sxg[k, pl.ds(v * L, L)]
            d_ids = dxg[k, pl.ds(v * L, L)]
            a = plsc.load_gather(asv, [s_ids]) + plsc.load_gather(adv, [d_ids])
            ex = jnp.exp(_lrelu(a) - mval)
            exc[pl.ds(v * L, L)] = ex
            for u in range(L):
                r = v * L + u
                av = _bcast_lane(ex, u)
                for q in range(D // L):
                    gp[r, pl.ds(q * L, L)] = gp[r, pl.ds(q * L, L)] * av
        pltpu.sync_copy(exc, dsh.at[dxg.at[k]], add=True)
        pltpu.sync_copy(gp, osh.at[dxg.at[k]], add=True)

    def _wait(gp, sem):
        pltpu.make_async_copy(h_hbm.at[pl.ds(0, CH)], gp, sem).wait()

    def _group(g, _):
        pltpu.sync_copy(sidx_hbm.at[g, wid], sxg)
        pltpu.sync_copy(didx_hbm.at[g, wid], dxg)
        pltpu.async_copy(h_hbm.at[sxg.at[0]], g0, sem0)

        def _pair(j, _):
            c0 = 2 * j
            pltpu.async_copy(h_hbm.at[sxg.at[c0 + 1]], g1, sem1)
            _wait(g0, sem0)
            _do_chunk(c0, g0)
            pltpu.async_copy(h_hbm.at[sxg.at[c0 + 2]], g0, sem0)
            _wait(g1, sem1)
            _do_chunk(c0 + 1, g1)
            return 0

        lax.fori_loop(0, NPAIR, _pair, 0)
        _wait(g0, sem0)
        _do_chunk(GG - 1, g0)
        return 0

    lax.fori_loop(0, NG, _group, 0)
    plsc.subcore_barrier()

    pltpu.sync_copy(dsh.at[pl.ds(sid * RPT, RPT)],
                    dcat_hbm.at[pl.ds(cid * NP + sid * RPT, RPT)])
    for j in range(NFULL):
        pltpu.sync_copy(osh.at[pl.ds(base + j * CH, CH)],
                        pcat_hbm.at[pl.ds(cid * NPR + base + j * CH, CH)])
    pltpu.sync_copy(
        osh.at[pl.ds(base + NFULL * CH, RPTR - NFULL * CH)],
        pcat_hbm.at[pl.ds(cid * NPR + base + NFULL * CH, RPTR - NFULL * CH)])


_sc_main = pl.kernel(
    _sc_main_body,
    out_type=[
        jax.ShapeDtypeStruct((2 * NP,), jnp.float32),
        jax.ShapeDtypeStruct((2 * NPR, D), jnp.float32),
    ],
    mesh=_MESH,
    compiler_params=_SC_PARAMS,
    scratch_types=[
        pltpu.VMEM((N,), jnp.float32),
        pltpu.VMEM((N,), jnp.float32),
        pltpu.VMEM((GG, CH), jnp.int32),
        pltpu.VMEM((GG, CH), jnp.int32),
        pltpu.VMEM((CH,), jnp.float32),
        pltpu.VMEM((16,), jnp.float32),
        pltpu.VMEM((CH, D), jnp.float32),
        pltpu.VMEM((CH, D), jnp.float32),
        pltpu.SemaphoreType.DMA,
        pltpu.SemaphoreType.DMA,
        pltpu.VMEM_SHARED((NP,), jnp.float32),
        pltpu.VMEM_SHARED((NPR, D), jnp.float32),
    ],
)


def _tc_post_body(pcat_ref, h_ref, as_ref, ad_ref, m_ref, dcat_ref,
                  b_ref, out_ref):
    a = _lrelu(as_ref[0:N] + ad_ref[0:N])
    se = jnp.exp(a - m_ref[0])
    dn = dcat_ref[0:N] + dcat_ref[NP:NP + N] + se
    num = (pcat_ref[0:N, :] + pcat_ref[NPR:NPR + N, :]
           + se[:, None] * h_ref[...])
    out_ref[...] = num / (dn + 1e-16)[:, None] + b_ref[...][None, :]


_tc_post = pl.pallas_call(
    _tc_post_body,
    out_shape=jax.ShapeDtypeStruct((N, D), jnp.float32),
)


def kernel(x, edge_index, edge_attr, W, att_src, att_dst, bias,
           edge_emb_weight):
    srcf = edge_index[0].astype(jnp.int32)
    dstf = edge_index[1].astype(jnp.int32)
    h, a_s, a_d, m = _tc_pre(x, W, att_src, att_dst)
    dcat, pcat = _sc_main(h, a_s, a_d, m,
                          srcf.reshape(NG, NW, GG, CH),
                          dstf.reshape(NG, NW, GG, CH))
    return _tc_post(pcat, h, a_s, a_d, m, dcat, bias)


# speedup vs baseline: 1.0881x; 1.0156x over previous
import functools

import jax
import jax.numpy as jnp
from jax import lax
from jax.experimental import pallas as pl
from jax.experimental.pallas import tpu as pltpu
from jax.experimental.pallas import tpu_sc as plsc

N = 10000
E = 320000
D = 128
NC, NS, L = 2, 16, 16
NW = NC * NS
EPT = E // NW
CH = 80
NCHK = EPT // CH
VPC = CH // L
NP = 10240
RPT = NP // NS
NPR = 10112
RPTR = NPR // NS

_MESH = plsc.VectorSubcoreMesh(
    core_axis_name="c", subcore_axis_name="s", num_cores=NC, num_subcores=NS)
_SC_PARAMS = pltpu.CompilerParams(needs_layout_passes=False)


def _lrelu(v):
    return jnp.where(v >= 0, v, 0.2 * v)


def _bcast_lane(vec, u):
    idx = jnp.full((L,), u, jnp.int32)
    return lax.gather(vec, idx[:, None],
                      dimension_numbers=lax.GatherDimensionNumbers(
                          offset_dims=(), collapsed_slice_dims=(0,),
                          start_index_map=(0,)),
                      slice_sizes=(1,),
                      mode=lax.GatherScatterMode.PROMISE_IN_BOUNDS)


def _tc_pre_body(x_ref, w_ref, asr_ref, adr_ref, h_ref, as_ref, ad_ref, m_ref):
    h = jnp.dot(x_ref[...], w_ref[...], preferred_element_type=jnp.float32)
    h_ref[...] = h
    a_s = jnp.sum(h * asr_ref[...][None, :], axis=1)
    a_d = jnp.sum(h * adr_ref[...][None, :], axis=1)
    pad = jnp.full((NP - N,), -1e30, jnp.float32)
    as_ref[...] = jnp.concatenate([a_s, pad])
    ad_ref[...] = jnp.concatenate([a_d, pad])
    m_ref[...] = jnp.full((128,), _lrelu(jnp.max(a_s) + jnp.max(a_d)),
                          jnp.float32)


_tc_pre = pl.pallas_call(
    _tc_pre_body,
    out_shape=[
        jax.ShapeDtypeStruct((N, D), jnp.float32),
        jax.ShapeDtypeStruct((NP,), jnp.float32),
        jax.ShapeDtypeStruct((NP,), jnp.float32),
        jax.ShapeDtypeStruct((128,), jnp.float32),
    ],
)


CH = 80
GG = 25
NG = NCHK // GG
NPAIR = (GG - 1) // 2
NFULL = RPTR // CH


def _sc_main_body(h_hbm, as_hbm, ad_hbm, m_hbm, sidx_hbm, didx_hbm,
                  dcat_hbm, pcat_hbm,
                  asv, adv, sxg, dxg, exc, mv, g0, g1, sem0, sem1, sem2,
                  dsh, osh):
    cid = lax.axis_index("c")
    sid = lax.axis_index("s")
    wid = cid * NS + sid

    pltpu.sync_copy(as_hbm.at[pl.ds(0, N)], asv)
    pltpu.sync_copy(ad_hbm.at[pl.ds(0, N)], adv)
    pltpu.sync_copy(m_hbm.at[pl.ds(0, 16)], mv)

    def _zg(r, _):
        for q in range(D // L):
            g0[r, pl.ds(q * L, L)] = jnp.zeros((L,), jnp.float32)
        return 0

    lax.fori_loop(0, CH, _zg, 0)
    base = sid * RPTR
    for j in range(NFULL):
        pltpu.sync_copy(g0, osh.at[pl.ds(base + j * CH, CH)])
    pltpu.sync_copy(g0.at[pl.ds(0, RPTR - NFULL * CH)],
                    osh.at[pl.ds(base + NFULL * CH, RPTR - NFULL * CH)])
    for j in range(RPT // 128):
        pltpu.sync_copy(g0.at[0, pl.ds(0, D)],
                        dsh.at[pl.ds(sid * RPT + j * 128, 128)])
    plsc.subcore_barrier()

    mval = mv[...]

    def _do_chunk(k, gp):
        @plsc.parallel_loop(0, VPC)
        def _vv(v):
            s_ids = sxg[k, pl.ds(v * L, L)]
            d_ids = dxg[k, pl.ds(v * L, L)]
            a = plsc.load_gather(asv, [s_ids]) + plsc.load_gather(adv, [d_ids])
            ex = jnp.exp(_lrelu(a) - mval)
            exc[pl.ds(v * L, L)] = ex
            for u in range(L):
                r = v * L + u
                av = _bcast_lane(ex, u)
                for q in range(D // L):
                    gp[r, pl.ds(q * L, L)] = gp[r, pl.ds(q * L, L)] * av
        pltpu.async_copy(exc, dsh.at[dxg.at[k]], sem2, add=True)
        pltpu.sync_copy(gp, osh.at[dxg.at[k]], add=True)
        pltpu.make_async_copy(as_hbm.at[pl.ds(0, CH)], exc, sem2).wait()

    def _wait(gp, sem):
        pltpu.make_async_copy(h_hbm.at[pl.ds(0, CH)], gp, sem).wait()

    def _group(g, _):
        pltpu.sync_copy(sidx_hbm.at[g, wid], sxg)
        pltpu.sync_copy(didx_hbm.at[g, wid], dxg)
        pltpu.async_copy(h_hbm.at[sxg.at[0]], g0, sem0)

        def _pair(j, _):
            c0 = 2 * j
            pltpu.async_copy(h_hbm.at[sxg.at[c0 + 1]], g1, sem1)
            _wait(g0, sem0)
            _do_chunk(c0, g0)
            pltpu.async_copy(h_hbm.at[sxg.at[c0 + 2]], g0, sem0)
            _wait(g1, sem1)
            _do_chunk(c0 + 1, g1)
            return 0

        lax.fori_loop(0, NPAIR, _pair, 0)
        _wait(g0, sem0)
        _do_chunk(GG - 1, g0)
        return 0

    lax.fori_loop(0, NG, _group, 0)
    plsc.subcore_barrier()

    pltpu.sync_copy(dsh.at[pl.ds(sid * RPT, RPT)],
                    dcat_hbm.at[pl.ds(cid * NP + sid * RPT, RPT)])
    for j in range(NFULL):
        pltpu.sync_copy(osh.at[pl.ds(base + j * CH, CH)],
                        pcat_hbm.at[pl.ds(cid * NPR + base + j * CH, CH)])
    pltpu.sync_copy(
        osh.at[pl.ds(base + NFULL * CH, RPTR - NFULL * CH)],
        pcat_hbm.at[pl.ds(cid * NPR + base + NFULL * CH, RPTR - NFULL * CH)])


_sc_main = pl.kernel(
    _sc_main_body,
    out_type=[
        jax.ShapeDtypeStruct((2 * NP,), jnp.float32),
        jax.ShapeDtypeStruct((2 * NPR, D), jnp.float32),
    ],
    mesh=_MESH,
    compiler_params=_SC_PARAMS,
    scratch_types=[
        pltpu.VMEM((N,), jnp.float32),
        pltpu.VMEM((N,), jnp.float32),
        pltpu.VMEM((GG, CH), jnp.int32),
        pltpu.VMEM((GG, CH), jnp.int32),
        pltpu.VMEM((CH,), jnp.float32),
        pltpu.VMEM((16,), jnp.float32),
        pltpu.VMEM((CH, D), jnp.float32),
        pltpu.VMEM((CH, D), jnp.float32),
        pltpu.SemaphoreType.DMA,
        pltpu.SemaphoreType.DMA,
        pltpu.SemaphoreType.DMA,
        pltpu.VMEM_SHARED((NP,), jnp.float32),
        pltpu.VMEM_SHARED((NPR, D), jnp.float32),
    ],
)


def _tc_post_body(pcat_ref, h_ref, as_ref, ad_ref, m_ref, dcat_ref,
                  b_ref, out_ref):
    a = _lrelu(as_ref[0:N] + ad_ref[0:N])
    se = jnp.exp(a - m_ref[0])
    dn = dcat_ref[0:N] + dcat_ref[NP:NP + N] + se
    num = (pcat_ref[0:N, :] + pcat_ref[NPR:NPR + N, :]
           + se[:, None] * h_ref[...])
    out_ref[...] = num / (dn + 1e-16)[:, None] + b_ref[...][None, :]


_tc_post = pl.pallas_call(
    _tc_post_body,
    out_shape=jax.ShapeDtypeStruct((N, D), jnp.float32),
)


def kernel(x, edge_index, edge_attr, W, att_src, att_dst, bias,
           edge_emb_weight):
    srcf = edge_index[0].astype(jnp.int32)
    dstf = edge_index[1].astype(jnp.int32)
    h, a_s, a_d, m = _tc_pre(x, W, att_src, att_dst)
    dcat, pcat = _sc_main(h, a_s, a_d, m,
                          srcf.reshape(NG, NW, GG, CH),
                          dstf.reshape(NG, NW, GG, CH))
    return _tc_post(pcat, h, a_s, a_d, m, dcat, bias)


# speedup vs baseline: 1.0887x; 1.0006x over previous
import jax
import jax.numpy as jnp
from jax import lax
from jax.experimental import pallas as pl
from jax.experimental.pallas import tpu as pltpu
from jax.experimental.pallas import tpu_sc as plsc

N = 10000
E = 320000
D = 128
NC, NS, L = 2, 16, 16
NW = NC * NS
EPT = E // NW
CH = 80
NCHK = EPT // CH
VPC = CH // L
NP = 10240
RPT = NP // NS
NPR = 10112
RPTR = NPR // NS

_MESH = plsc.VectorSubcoreMesh(
    core_axis_name="c", subcore_axis_name="s", num_cores=NC, num_subcores=NS)
_SC_PARAMS = pltpu.CompilerParams(needs_layout_passes=False)


def _lrelu(v):
    return jnp.where(v >= 0, v, 0.2 * v)


def _bcast_lane(vec, u):
    idx = jnp.full((L,), u, jnp.int32)
    return lax.gather(vec, idx[:, None],
                      dimension_numbers=lax.GatherDimensionNumbers(
                          offset_dims=(), collapsed_slice_dims=(0,),
                          start_index_map=(0,)),
                      slice_sizes=(1,),
                      mode=lax.GatherScatterMode.PROMISE_IN_BOUNDS)


def _tc_pre_body(x_ref, w_ref, asr_ref, adr_ref, h_ref, as_ref, ad_ref, m_ref):
    h = jnp.dot(x_ref[...], w_ref[...], preferred_element_type=jnp.float32)
    h_ref[...] = h
    a_s = jnp.sum(h * asr_ref[...][None, :], axis=1)
    a_d = jnp.sum(h * adr_ref[...][None, :], axis=1)
    pad = jnp.full((NP - N,), -1e30, jnp.float32)
    as_ref[...] = jnp.concatenate([a_s, pad])
    ad_ref[...] = jnp.concatenate([a_d, pad])
    m_ref[...] = jnp.full((128,), _lrelu(jnp.max(a_s) + jnp.max(a_d)),
                          jnp.float32)


_tc_pre = pl.pallas_call(
    _tc_pre_body,
    out_shape=[
        jax.ShapeDtypeStruct((N, D), jnp.float32),
        jax.ShapeDtypeStruct((NP,), jnp.float32),
        jax.ShapeDtypeStruct((NP,), jnp.float32),
        jax.ShapeDtypeStruct((128,), jnp.float32),
    ],
)


CH = 80
GG = 25
NG = NCHK // GG
NPAIR = (GG - 1) // 2
NFULL = RPTR // CH


def _sc_main_body(h_hbm, as_hbm, ad_hbm, m_hbm, sidx_hbm, didx_hbm,
                  dcat_hbm, pcat_hbm,
                  asv, adv, sxg, dxg, exc, mv, g0, g1, sem0, sem1, sem2,
                  dsh, osh):
    cid = lax.axis_index("c")
    sid = lax.axis_index("s")
    wid = cid * NS + sid

    pltpu.sync_copy(as_hbm.at[pl.ds(0, N)], asv)
    pltpu.sync_copy(ad_hbm.at[pl.ds(0, N)], adv)
    pltpu.sync_copy(m_hbm.at[pl.ds(0, 16)], mv)

    def _zg(r, _):
        for q in range(D // L):
            g0[r, pl.ds(q * L, L)] = jnp.zeros((L,), jnp.float32)
        return 0

    lax.fori_loop(0, CH, _zg, 0)
    base = sid * RPTR
    for j in range(NFULL):
        pltpu.sync_copy(g0, osh.at[pl.ds(base + j * CH, CH)])
    pltpu.sync_copy(g0.at[pl.ds(0, RPTR - NFULL * CH)],
                    osh.at[pl.ds(base + NFULL * CH, RPTR - NFULL * CH)])
    for j in range(RPT // 128):
        pltpu.sync_copy(g0.at[0, pl.ds(0, D)],
                        dsh.at[pl.ds(sid * RPT + j * 128, 128)])
    plsc.subcore_barrier()

    mval = mv[...]

    def _do_chunk(k, gp):
        @plsc.parallel_loop(0, VPC)
        def _vv(v):
            s_ids = sxg[k, pl.ds(v * L, L)]
            d_ids = dxg[k, pl.ds(v * L, L)]
            a = plsc.load_gather(asv, [s_ids]) + plsc.load_gather(adv, [d_ids])
            ex = jnp.exp(_lrelu(a) - mval)
            exc[pl.ds(v * L, L)] = ex
            for u in range(L):
                r = v * L + u
                av = _bcast_lane(ex, u)
                for q in range(D // L):
                    gp[r, pl.ds(q * L, L)] = gp[r, pl.ds(q * L, L)] * av
        pltpu.async_copy(exc, dsh.at[dxg.at[k]], sem2, add=True)
        pltpu.sync_copy(gp, osh.at[dxg.at[k]], add=True)
        pltpu.make_async_copy(as_hbm.at[pl.ds(0, CH)], exc, sem2).wait()

    def _wait(gp, sem):
        pltpu.make_async_copy(h_hbm.at[pl.ds(0, CH)], gp, sem).wait()

    def _group(g, _):
        pltpu.sync_copy(sidx_hbm.at[g, wid], sxg)
        pltpu.sync_copy(didx_hbm.at[g, wid], dxg)
        pltpu.async_copy(h_hbm.at[sxg.at[0]], g0, sem0)

        def _pair(j, _):
            c0 = 2 * j
            pltpu.async_copy(h_hbm.at[sxg.at[c0 + 1]], g1, sem1)
            _wait(g0, sem0)
            _do_chunk(c0, g0)
            pltpu.async_copy(h_hbm.at[sxg.at[c0 + 2]], g0, sem0)
            _wait(g1, sem1)
            _do_chunk(c0 + 1, g1)
            return 0

        lax.fori_loop(0, NPAIR, _pair, 0)
        _wait(g0, sem0)
        _do_chunk(GG - 1, g0)
        return 0

    lax.fori_loop(0, NG, _group, 0)
    plsc.subcore_barrier()

    pltpu.sync_copy(dsh.at[pl.ds(sid * RPT, RPT)],
                    dcat_hbm.at[pl.ds(cid * NP + sid * RPT, RPT)])
    for j in range(NFULL):
        pltpu.sync_copy(osh.at[pl.ds(base + j * CH, CH)],
                        pcat_hbm.at[pl.ds(cid * NPR + base + j * CH, CH)])
    pltpu.sync_copy(
        osh.at[pl.ds(base + NFULL * CH, RPTR - NFULL * CH)],
        pcat_hbm.at[pl.ds(cid * NPR + base + NFULL * CH, RPTR - NFULL * CH)])


_sc_main = pl.kernel(
    _sc_main_body,
    out_type=[
        jax.ShapeDtypeStruct((2 * NP,), jnp.float32),
        jax.ShapeDtypeStruct((2 * NPR, D), jnp.float32),
    ],
    mesh=_MESH,
    compiler_params=_SC_PARAMS,
    scratch_types=[
        pltpu.VMEM((N,), jnp.float32),
        pltpu.VMEM((N,), jnp.float32),
        pltpu.VMEM((GG, CH), jnp.int32),
        pltpu.VMEM((GG, CH), jnp.int32),
        pltpu.VMEM((CH,), jnp.float32),
        pltpu.VMEM((16,), jnp.float32),
        pltpu.VMEM((CH, D), jnp.float32),
        pltpu.VMEM((CH, D), jnp.float32),
        pltpu.SemaphoreType.DMA,
        pltpu.SemaphoreType.DMA,
        pltpu.SemaphoreType.DMA,
        pltpu.VMEM_SHARED((NP,), jnp.float32),
        pltpu.VMEM_SHARED((NPR, D), jnp.float32),
    ],
)


def _tc_post_body(pcat_ref, h_ref, as_ref, ad_ref, m_ref, dcat_ref,
                  b_ref, out_ref):
    a = _lrelu(as_ref[0:N] + ad_ref[0:N])
    se = jnp.exp(a - m_ref[0])
    dn = dcat_ref[0:N] + dcat_ref[NP:NP + N] + se
    num = (pcat_ref[0:N, :] + pcat_ref[NPR:NPR + N, :]
           + se[:, None] * h_ref[...])
    out_ref[...] = num / (dn + 1e-16)[:, None] + b_ref[...][None, :]


_tc_post = pl.pallas_call(
    _tc_post_body,
    out_shape=jax.ShapeDtypeStruct((N, D), jnp.float32),
)


def kernel(x, edge_index, edge_attr, W, att_src, att_dst, bias,
           edge_emb_weight):
    srcf = edge_index[0].astype(jnp.int32)
    dstf = edge_index[1].astype(jnp.int32)
    h, a_s, a_d, m = _tc_pre(x, W, att_src, att_dst)
    dcat, pcat = _sc_main(h, a_s, a_d, m,
                          srcf.reshape(NG, NW, GG, CH),
                          dstf.reshape(NG, NW, GG, CH))
    return _tc_post(pcat, h, a_s, a_d, m, dcat, bias)
